# Initial kernel scaffold; baseline (speedup 1.0000x reference)
import jax
import jax.numpy as jnp
from jax.experimental import pallas as pl


def kernel(title_ids, token_ids, title_table, text_table):
    raise NotImplementedError("write your pallas kernel here")


# speedup vs baseline: 9.1356x; 9.1356x over previous
import functools

import jax
import jax.numpy as jnp
from jax import lax
from jax.experimental import pallas as pl
from jax.experimental.pallas import tpu as pltpu
from jax.experimental.pallas import tpu_sc as plsc

B = 16384
L = 200
LP = 208
DIM = 32
NW = 32
BPW = B // NW
CB = 32
NCH = BPW // CB


def _sc_kernel(title_h, tok_h, ttab_h, xtab_h, out_h,
               ids_v, rows_v, tids_v, trows_v, outc_v, cnt_v, sem):
    wid = lax.axis_index("s") * 2 + lax.axis_index("c")
    base = wid * BPW
    lanes = lax.iota(jnp.int32, 16)

    def chunk_body(c, _):
        cb = base + c * CB
        pltpu.sync_copy(tok_h.at[pl.ds(cb * LP, CB * LP)], ids_v)
        pltpu.sync_copy(title_h.at[pl.ds(cb, CB)], tids_v)
        pltpu.async_copy(ttab_h.at[tids_v], trows_v, sem).wait()

        def cnt_grp(g, _):
            rows16 = g * 16 + lanes

            def cnt_col(j, cnt):
                ids16 = plsc.load_gather(ids_v, [rows16 * LP + j])
                return cnt + jnp.where(ids16 != 0, 1.0, 0.0)

            cnt = lax.fori_loop(0, L, cnt_col, jnp.zeros((16,), jnp.float32))
            cnt_v[pl.ds(g * 16, 16)] = jnp.maximum(cnt, 1.0)
            return 0

        lax.fori_loop(0, CB // 16, cnt_grp, 0)

        def row_body(r, _):
            pltpu.async_copy(
                xtab_h.at[ids_v.at[pl.ds(r * LP, LP)]], rows_v, sem).wait()
            zero = jnp.zeros((16,), jnp.float32)

            def acc_body(j, carry):
                a0, a1 = carry
                a0 = a0 + rows_v[j, pl.ds(0, 16)]
                a1 = a1 + rows_v[j, pl.ds(16, 16)]
                return a0, a1

            a0, a1 = lax.fori_loop(0, LP, acc_body, (zero, zero))
            denom = plsc.load_gather(cnt_v, [jnp.full((16,), 1, jnp.int32) * r])
            outc_v[r, pl.ds(0, 16)] = trows_v[r, pl.ds(0, 16)]
            outc_v[r, pl.ds(16, 16)] = trows_v[r, pl.ds(16, 16)]
            outc_v[r, pl.ds(32, 16)] = a0 / denom
            outc_v[r, pl.ds(48, 16)] = a1 / denom
            return 0

        lax.fori_loop(0, CB, row_body, 0)
        pltpu.sync_copy(outc_v, out_h.at[pl.ds(cb, CB)])
        return 0

    lax.fori_loop(0, NCH, chunk_body, 0)


def kernel(title_ids, token_ids, title_table, text_table):
    text_z = text_table.at[0].set(0.0)
    tok_pad = jnp.pad(token_ids, ((0, 0), (0, LP - L))).reshape(B * LP)

    mesh = plsc.VectorSubcoreMesh(core_axis_name="c", subcore_axis_name="s")
    run = functools.partial(
        pl.kernel,
        mesh=mesh,
        compiler_params=pltpu.CompilerParams(
            needs_layout_passes=False, use_tc_tiling_on_sc=False),
        out_type=jax.ShapeDtypeStruct((B, 2 * DIM), jnp.float32),
        scratch_types=[
            pltpu.VMEM((CB * LP,), jnp.int32),
            pltpu.VMEM((LP, DIM), jnp.float32),
            pltpu.VMEM((CB,), jnp.int32),
            pltpu.VMEM((CB, DIM), jnp.float32),
            pltpu.VMEM((CB, 2 * DIM), jnp.float32),
            pltpu.VMEM((CB,), jnp.float32),
            pltpu.SemaphoreType.DMA,
        ],
    )(_sc_kernel)
    return run(title_ids, tok_pad, title_table, text_z)


# speedup vs baseline: 9.2171x; 1.0089x over previous
import functools

import jax
import jax.numpy as jnp
from jax import lax
from jax.experimental import pallas as pl
from jax.experimental.pallas import tpu as pltpu
from jax.experimental.pallas import tpu_sc as plsc

B = 16384
L = 200
LP = 208
DIM = 32
NW = 32
BPW = B // NW
CB = 64
NCH = BPW // CB
NBUF = 4


def _sc_kernel(title_h, tok_h, ttab_h, xtab_h, out_h,
               ids_v, rows_v, tids_v, trows_v, outc_v, cnt_v,
               tsem, *sems):
    wid = lax.axis_index("s") * 2 + lax.axis_index("c")
    base = wid * BPW
    lanes = lax.iota(jnp.int32, 16)

    def tok_gather(r, b):
        return pltpu.make_async_copy(
            xtab_h.at[ids_v.at[pl.ds(r * LP, LP)]],
            rows_v.at[b], sems[b])

    def chunk_body(c, _):
        cb = base + c * CB
        pltpu.sync_copy(tok_h.at[pl.ds(cb * LP, CB * LP)], ids_v)
        pltpu.sync_copy(title_h.at[pl.ds(cb, CB)], tids_v)
        pltpu.make_async_copy(ttab_h.at[tids_v], trows_v, tsem).start()
        for b in range(NBUF):
            tok_gather(b, b).start()

        def cnt_grp(g, _):
            rowbase = (g * 16 + lanes) * LP

            def cnt_col(j, cnt):
                for k in range(8):
                    ids16 = plsc.load_gather(ids_v, [rowbase + (j * 8 + k)])
                    cnt = cnt + jnp.where(ids16 != 0, 1.0, 0.0)
                return cnt

            cnt = lax.fori_loop(0, L // 8, cnt_col,
                                jnp.zeros((16,), jnp.float32))
            cnt_v[pl.ds(g * 16, 16)] = jnp.maximum(cnt, 1.0)
            return 0

        lax.fori_loop(0, CB // 16, cnt_grp, 0)
        pltpu.make_async_copy(ttab_h.at[tids_v], trows_v, tsem).wait()

        def grp_body(g, _):
            for b in range(NBUF):
                r = g * NBUF + b
                tok_gather(r, b).wait()
                zero = jnp.zeros((16,), jnp.float32)

                def acc_body(j, carry):
                    a0, a1, a2, a3 = carry
                    for k in range(8):
                        t = j * 16 + 2 * k
                        a0 = a0 + rows_v[b, t, pl.ds(0, 16)]
                        a1 = a1 + rows_v[b, t, pl.ds(16, 16)]
                        a2 = a2 + rows_v[b, t + 1, pl.ds(0, 16)]
                        a3 = a3 + rows_v[b, t + 1, pl.ds(16, 16)]
                    return a0, a1, a2, a3

                a0, a1, a2, a3 = lax.fori_loop(0, LP // 16, acc_body,
                                               (zero, zero, zero, zero))

                @pl.when(r + NBUF < CB)
                def _():
                    tok_gather(r + NBUF, b).start()

                denom = plsc.load_gather(
                    cnt_v, [jnp.full((16,), 1, jnp.int32) * r])
                outc_v[r, pl.ds(0, 16)] = trows_v[r, pl.ds(0, 16)]
                outc_v[r, pl.ds(16, 16)] = trows_v[r, pl.ds(16, 16)]
                outc_v[r, pl.ds(32, 16)] = (a0 + a2) / denom
                outc_v[r, pl.ds(48, 16)] = (a1 + a3) / denom
            return 0

        lax.fori_loop(0, CB // NBUF, grp_body, 0)
        pltpu.sync_copy(outc_v, out_h.at[pl.ds(cb, CB)])
        return 0

    lax.fori_loop(0, NCH, chunk_body, 0)


def kernel(title_ids, token_ids, title_table, text_table):
    text_z = text_table.at[0].set(0.0)
    tok_pad = jnp.pad(token_ids, ((0, 0), (0, LP - L))).reshape(B * LP)

    mesh = plsc.VectorSubcoreMesh(core_axis_name="c", subcore_axis_name="s")
    run = functools.partial(
        pl.kernel,
        mesh=mesh,
        compiler_params=pltpu.CompilerParams(
            needs_layout_passes=False, use_tc_tiling_on_sc=False),
        out_type=jax.ShapeDtypeStruct((B, 2 * DIM), jnp.float32),
        scratch_types=[
            pltpu.VMEM((CB * LP,), jnp.int32),
            pltpu.VMEM((NBUF, LP, DIM), jnp.float32),
            pltpu.VMEM((CB,), jnp.int32),
            pltpu.VMEM((CB, DIM), jnp.float32),
            pltpu.VMEM((CB, 2 * DIM), jnp.float32),
            pltpu.VMEM((CB,), jnp.float32),
            pltpu.SemaphoreType.DMA,
        ] + [pltpu.SemaphoreType.DMA] * NBUF,
    )(_sc_kernel)
    return run(title_ids, tok_pad, title_table, text_z)
